# baseline (speedup 1.0000x reference)
import functools

import jax
import jax.numpy as jnp
from jax import lax
from jax.experimental import pallas as pl
from jax.experimental.pallas import tpu as pltpu
from jax.experimental.pallas import tpu_sc as plsc

NUM_CORES = 2
NUM_SUBCORES = 16
NUM_WORKERS = NUM_CORES * NUM_SUBCORES
LOOKUP_ROWS = 24


def _sc_lookup(N, D, pos_start, t_hbm, emb_hbm, row_v):
    wid = lax.axis_index("s") * NUM_CORES + lax.axis_index("c")
    nw = N // LOOKUP_ROWS

    @pl.when(wid < nw)
    def _():
        r0 = wid * LOOKUP_ROWS
        pltpu.sync_copy(t_hbm.at[pl.ds(pos_start + r0, LOOKUP_ROWS)], row_v)
        pltpu.sync_copy(row_v, emb_hbm.at[pl.ds(r0, LOOKUP_ROWS)])


def _tc_add_body(p_ref, t_ref, o_ref):
    o_ref[...] = p_ref[...] + t_ref[...]


def kernel(patches, pos_table):
    B, N, D = patches.shape
    pos_start = pos_table.shape[0] - N

    mesh = plsc.VectorSubcoreMesh(core_axis_name="c", subcore_axis_name="s")
    lookup = functools.partial(
        pl.kernel,
        out_type=jax.ShapeDtypeStruct((N, D), pos_table.dtype),
        mesh=mesh,
        scratch_types=[pltpu.VMEM((LOOKUP_ROWS, D), pos_table.dtype)],
    )(functools.partial(_sc_lookup, N, D, pos_start))
    pos_emb = lookup(pos_table)

    BB = 8
    return pl.pallas_call(
        _tc_add_body,
        grid=(B // BB,),
        in_specs=[
            pl.BlockSpec((BB, N, D), lambda i: (i, 0, 0)),
            pl.BlockSpec((N, D), lambda i: (0, 0)),
        ],
        out_specs=pl.BlockSpec((BB, N, D), lambda i: (i, 0, 0)),
        out_shape=jax.ShapeDtypeStruct((B, N, D), patches.dtype),
    )(patches, pos_emb)
